# Initial kernel scaffold; baseline (speedup 1.0000x reference)
import jax
import jax.numpy as jnp
from jax.experimental import pallas as pl


def kernel(x, edge_index, W1, b1, W_mu, b_mu, W_ls, b_ls):
    raise NotImplementedError("write your pallas kernel here")


# speedup vs baseline: 19.0966x; 19.0966x over previous
import functools

import jax
import jax.numpy as jnp
from jax import lax
from jax.experimental import pallas as pl
from jax.experimental.pallas import tpu as pltpu
from jax.experimental.pallas import tpu_sc as plsc

_NC = 2
_NS = 16
_NW = _NC * _NS
_IB = 128


def _edge_range(wid, nrows):
    per = nrows // _NW
    rem = nrows % _NW
    base = wid * per + jnp.minimum(wid, rem)
    cnt = per + jnp.where(wid < rem, 1, 0)
    return base, cnt


def _deg_body(dst2d, out, ibuf, ones_v, zbuf, accum):
    c = lax.axis_index("c")
    s = lax.axis_index("s")
    wid = c * _NS + s
    npad = accum.shape[0]
    rpt = npad // _NS

    def fill_ones(i, _):
        ones_v[pl.ds(i * 16, 16)] = jnp.ones((16,), jnp.float32)
        return 0

    lax.fori_loop(0, _IB // 16, fill_ones, 0)

    def fill_zero(i, _):
        zbuf[pl.ds(i * 16, 16)] = jnp.zeros((16,), jnp.float32)
        return 0

    lax.fori_loop(0, rpt // 16, fill_zero, 0)

    pltpu.sync_copy(zbuf, accum.at[pl.ds(s * rpt, rpt)])
    plsc.subcore_barrier()

    base, cnt = _edge_range(wid, dst2d.shape[0])

    def body(j, _):
        pltpu.sync_copy(dst2d.at[base + j], ibuf.at[0])
        pltpu.sync_copy(ones_v, accum.at[ibuf.at[0]], add=True)
        return 0

    lax.fori_loop(0, cnt, body, 0)
    plsc.subcore_barrier()
    pltpu.sync_copy(accum.at[pl.ds(s * rpt, rpt)], out.at[c, pl.ds(s * rpt, rpt)])


def _make_deg(nrows, npad):
    mesh = plsc.VectorSubcoreMesh(core_axis_name="c", subcore_axis_name="s")
    return pl.kernel(
        _deg_body,
        out_type=jax.ShapeDtypeStruct((_NC, npad), jnp.float32),
        mesh=mesh,
        scratch_types=[
            pltpu.VMEM((1, _IB), jnp.int32),
            pltpu.VMEM((_IB,), jnp.float32),
            pltpu.VMEM((npad // _NS,), jnp.float32),
            pltpu.VMEM_SHARED((npad,), jnp.float32),
        ],
    )


def _agg_body(hs, src2d, dst2d, out, ibuf, rows_v, accum, sem):
    c = lax.axis_index("c")
    s = lax.axis_index("s")
    wid = c * _NS + s
    n = accum.shape[0]
    rpt = n // _NS

    def zfill(i, _):
        for cb in range(8):
            rows_v[i, pl.ds(cb * 16, 16)] = jnp.zeros((16,), jnp.float32)
        return 0

    lax.fori_loop(0, _IB, zfill, 0)

    full, rem = rpt // _IB, rpt % _IB
    for i in range(full):
        pltpu.sync_copy(rows_v, accum.at[pl.ds(s * rpt + i * _IB, _IB)])
    if rem:
        pltpu.sync_copy(
            rows_v.at[pl.ds(0, rem)],
            accum.at[pl.ds(s * rpt + full * _IB, rem)],
        )
    plsc.subcore_barrier()

    base, cnt = _edge_range(wid, src2d.shape[0])

    def body(j, _):
        pltpu.sync_copy(src2d.at[base + j], ibuf.at[0])
        pltpu.sync_copy(dst2d.at[base + j], ibuf.at[1])
        pltpu.async_copy(hs.at[ibuf.at[0]], rows_v, sem).wait()
        pltpu.sync_copy(rows_v, accum.at[ibuf.at[1]], add=True)
        return 0

    lax.fori_loop(0, cnt, body, 0)
    plsc.subcore_barrier()
    pltpu.sync_copy(
        accum.at[pl.ds(s * rpt, rpt)], out.at[c, pl.ds(s * rpt, rpt)]
    )


def _make_agg(npad, d, nrows):
    mesh = plsc.VectorSubcoreMesh(core_axis_name="c", subcore_axis_name="s")
    return pl.kernel(
        _agg_body,
        out_type=jax.ShapeDtypeStruct((_NC, npad, d), jnp.float32),
        mesh=mesh,
        scratch_types=[
            pltpu.VMEM((2, _IB), jnp.int32),
            pltpu.VMEM((_IB, d), jnp.float32),
            pltpu.VMEM_SHARED((npad, d), jnp.float32),
            pltpu.SemaphoreType.DMA,
        ],
    )


def _b1_body(degp_ref, x_ref, w1_ref, hs_ref, dis_ref):
    deg = degp_ref[0] + degp_ref[1] + 1.0
    dis = lax.rsqrt(deg)
    h1 = jnp.dot(x_ref[...], w1_ref[...], preferred_element_type=jnp.float32)
    hs_ref[...] = h1 * dis
    dis_ref[...] = dis


def _b2_body(pp_ref, dis_ref, hs1_ref, w2_ref, b1_ref, hs2_ref):
    dis = dis_ref[...]
    agg = pp_ref[0] + pp_ref[1]
    h = jnp.maximum(dis * agg + dis * hs1_ref[...] + b1_ref[...], 0.0)
    h2 = jnp.dot(h, w2_ref[...], preferred_element_type=jnp.float32)
    hs2_ref[...] = h2 * dis


def _b3_body(qq_ref, dis_ref, hs2_ref, b2_ref, out_ref):
    dis = dis_ref[...]
    out_ref[...] = dis * (qq_ref[0] + qq_ref[1]) + dis * hs2_ref[...] + b2_ref[...]


def kernel(x, edge_index, W1, b1, W_mu, b_mu, W_ls, b_ls):
    n, d_in = x.shape
    e = edge_index.shape[1]
    d_hid = W1.shape[1]
    d_out = W_mu.shape[1]

    nrows = e // _IB
    src2d = edge_index[0].reshape(nrows, _IB)
    dst2d = edge_index[1].reshape(nrows, _IB)
    npad = ((n + 255) // 256) * 256

    w2 = jnp.concatenate([W_mu, W_ls], axis=1)
    b1r = b1.reshape(1, d_hid)
    b2r = jnp.concatenate([b_mu, b_ls]).reshape(1, 2 * d_out)

    degp = _make_deg(nrows, npad)(dst2d)
    degp3 = degp[:, :n].reshape(_NC, n, 1)

    r = 1000 if n % 1000 == 0 else 8 * (n // 8)
    grid = (n // r,)
    f32 = jnp.float32

    hs1, dis = pl.pallas_call(
        _b1_body,
        grid=grid,
        in_specs=[
            pl.BlockSpec((_NC, r, 1), lambda i: (0, i, 0)),
            pl.BlockSpec((r, d_in), lambda i: (i, 0)),
            pl.BlockSpec((d_in, d_hid), lambda i: (0, 0)),
        ],
        out_specs=[
            pl.BlockSpec((r, d_hid), lambda i: (i, 0)),
            pl.BlockSpec((r, 1), lambda i: (i, 0)),
        ],
        out_shape=[
            jax.ShapeDtypeStruct((n, d_hid), f32),
            jax.ShapeDtypeStruct((n, 1), f32),
        ],
    )(degp3, x, W1)

    pp = _make_agg(npad, d_hid, nrows)(hs1, src2d, dst2d)[:, :n]

    hs2 = pl.pallas_call(
        _b2_body,
        grid=grid,
        in_specs=[
            pl.BlockSpec((_NC, r, d_hid), lambda i: (0, i, 0)),
            pl.BlockSpec((r, 1), lambda i: (i, 0)),
            pl.BlockSpec((r, d_hid), lambda i: (i, 0)),
            pl.BlockSpec((d_hid, 2 * d_out), lambda i: (0, 0)),
            pl.BlockSpec((1, d_hid), lambda i: (0, 0)),
        ],
        out_specs=pl.BlockSpec((r, 2 * d_out), lambda i: (i, 0)),
        out_shape=jax.ShapeDtypeStruct((n, 2 * d_out), f32),
    )(pp, dis, hs1, w2, b1r)

    qq = _make_agg(npad, 2 * d_out, nrows)(hs2, src2d, dst2d)[:, :n]

    out = pl.pallas_call(
        _b3_body,
        grid=grid,
        in_specs=[
            pl.BlockSpec((_NC, r, 2 * d_out), lambda i: (0, i, 0)),
            pl.BlockSpec((r, 1), lambda i: (i, 0)),
            pl.BlockSpec((r, 2 * d_out), lambda i: (i, 0)),
            pl.BlockSpec((1, 2 * d_out), lambda i: (0, 0)),
        ],
        out_specs=pl.BlockSpec((r, 2 * d_out), lambda i: (i, 0)),
        out_shape=jax.ShapeDtypeStruct((n, 2 * d_out), f32),
    )(qq, dis, hs2, b2r)

    return (out[:, :d_out], out[:, d_out:])


# speedup vs baseline: 36.3834x; 1.9052x over previous
import functools

import jax
import jax.numpy as jnp
from jax import lax
from jax.experimental import pallas as pl
from jax.experimental.pallas import tpu as pltpu
from jax.experimental.pallas import tpu_sc as plsc

_NC = 2
_NS = 16
_NW = _NC * _NS
_IB = 128


_NBUF = 2


def _deg_body(dst2d, out, didx, ones_v, zbuf, accum):
    c = lax.axis_index("c")
    s = lax.axis_index("s")
    wid = c * _NS + s
    npad = accum.shape[0]
    rpt = npad // _NS
    rows = dst2d.shape[0] // _NW

    def fill_ones(i, _):
        ones_v[pl.ds(i * 16, 16)] = jnp.ones((16,), jnp.float32)
        return 0

    lax.fori_loop(0, _IB // 16, fill_ones, 0)

    def fill_zero(i, _):
        zbuf[pl.ds(i * 16, 16)] = jnp.zeros((16,), jnp.float32)
        return 0

    lax.fori_loop(0, rpt // 16, fill_zero, 0)

    pltpu.sync_copy(zbuf, accum.at[pl.ds(s * rpt, rpt)])
    pltpu.sync_copy(dst2d.at[pl.ds(wid * rows, rows)], didx)
    plsc.subcore_barrier()

    def body(j, _):
        pltpu.sync_copy(ones_v, accum.at[didx.at[j]], add=True)
        return 0

    lax.fori_loop(0, rows, body, 0)
    plsc.subcore_barrier()
    pltpu.sync_copy(accum.at[pl.ds(s * rpt, rpt)], out.at[c, pl.ds(s * rpt, rpt)])


def _make_deg(nrows, npad):
    mesh = plsc.VectorSubcoreMesh(core_axis_name="c", subcore_axis_name="s")
    return pl.kernel(
        _deg_body,
        out_type=jax.ShapeDtypeStruct((_NC, npad), jnp.float32),
        mesh=mesh,
        scratch_types=[
            pltpu.VMEM((nrows // _NW, _IB), jnp.int32),
            pltpu.VMEM((_IB,), jnp.float32),
            pltpu.VMEM((npad // _NS,), jnp.float32),
            pltpu.VMEM_SHARED((npad,), jnp.float32),
        ],
    )


_CH = 16


def _agg_body(hs, src2d, dst2d, out, sc0, sc1, dc0, dc1, b0, b1, accum, s0, s1):
    c = lax.axis_index("c")
    s = lax.axis_index("s")
    wid = c * _NS + s
    n = accum.shape[0]
    rpt = n // _NS
    rows = src2d.shape[0] // _NW
    nch = rows // _CH
    base = wid * rows
    scb = (sc0, sc1)
    dcb = (dc0, dc1)
    bufs = (b0, b1)
    sems = (s0, s1)

    def zfill(i, _):
        for cb in range(8):
            b0[i, pl.ds(cb * 16, 16)] = jnp.zeros((16,), jnp.float32)
        return 0

    lax.fori_loop(0, _IB, zfill, 0)
    for i in range(rpt // _IB):
        pltpu.sync_copy(b0, accum.at[pl.ds(s * rpt + i * _IB, _IB)])
    plsc.subcore_barrier()

    pltpu.sync_copy(src2d.at[pl.ds(base, _CH)], sc0)
    pltpu.sync_copy(dst2d.at[pl.ds(base, _CH)], dc0)
    pltpu.async_copy(hs.at[sc0.at[0]], b0, s0)
    pltpu.async_copy(hs.at[sc0.at[1]], b1, s1)

    for k in range(nch):
        cur_s, cur_d = scb[k % 2], dcb[k % 2]
        nxt_s, nxt_d = scb[(k + 1) % 2], dcb[(k + 1) % 2]
        if k + 1 < nch:
            pltpu.sync_copy(src2d.at[pl.ds(base + (k + 1) * _CH, _CH)], nxt_s)
            pltpu.sync_copy(dst2d.at[pl.ds(base + (k + 1) * _CH, _CH)], nxt_d)
        for jj in range(_CH):
            j = k * _CH + jj
            b = jj % 2
            pltpu.make_async_copy(hs.at[cur_s.at[jj]], bufs[b], sems[b]).wait()
            pltpu.sync_copy(bufs[b], accum.at[cur_d.at[jj]], add=True)
            nj = jj + 2
            if j + 2 < rows:
                if nj < _CH:
                    pltpu.async_copy(hs.at[cur_s.at[nj]], bufs[b], sems[b])
                else:
                    pltpu.async_copy(hs.at[nxt_s.at[nj - _CH]], bufs[b], sems[b])

    plsc.subcore_barrier()
    pltpu.sync_copy(
        accum.at[pl.ds(s * rpt, rpt)], out.at[c, pl.ds(s * rpt, rpt)]
    )


def _make_agg(npad, d, nrows):
    mesh = plsc.VectorSubcoreMesh(core_axis_name="c", subcore_axis_name="s")
    return pl.kernel(
        _agg_body,
        out_type=jax.ShapeDtypeStruct((_NC, npad, d), jnp.float32),
        mesh=mesh,
        scratch_types=[
            pltpu.VMEM((_CH, _IB), jnp.int32),
            pltpu.VMEM((_CH, _IB), jnp.int32),
            pltpu.VMEM((_CH, _IB), jnp.int32),
            pltpu.VMEM((_CH, _IB), jnp.int32),
            pltpu.VMEM((_IB, d), jnp.float32),
            pltpu.VMEM((_IB, d), jnp.float32),
            pltpu.VMEM_SHARED((npad, d), jnp.float32),
            pltpu.SemaphoreType.DMA,
            pltpu.SemaphoreType.DMA,
        ],
    )


def _b1_body(degp_ref, x_ref, w1_ref, hs_ref, dis_ref):
    deg = degp_ref[0] + degp_ref[1] + 1.0
    dis = lax.rsqrt(deg)
    h1 = jnp.dot(x_ref[...], w1_ref[...], preferred_element_type=jnp.float32)
    hs_ref[...] = h1 * dis
    dis_ref[...] = dis


def _b2_body(pp_ref, dis_ref, hs1_ref, w2_ref, b1_ref, hs2_ref):
    dis = dis_ref[...]
    agg = pp_ref[0] + pp_ref[1]
    h = jnp.maximum(dis * agg + dis * hs1_ref[...] + b1_ref[...], 0.0)
    h2 = jnp.dot(h, w2_ref[...], preferred_element_type=jnp.float32)
    hs2_ref[...] = h2 * dis


def _b3_body(qq_ref, dis_ref, hs2_ref, b2_ref, out_ref):
    dis = dis_ref[...]
    out_ref[...] = dis * (qq_ref[0] + qq_ref[1]) + dis * hs2_ref[...] + b2_ref[...]


def kernel(x, edge_index, W1, b1, W_mu, b_mu, W_ls, b_ls):
    n, d_in = x.shape
    e = edge_index.shape[1]
    d_hid = W1.shape[1]
    d_out = W_mu.shape[1]

    npad = ((n + 255) // 256) * 256

    rows_pt = 8 * (-(-e // (_IB * _NW * 8)))
    nrows = _NW * rows_pt
    pad = nrows * _IB - e
    ar = jnp.arange(pad, dtype=jnp.int32)
    src_full = jnp.concatenate([edge_index[0], ar % n])
    dst_full = jnp.concatenate([edge_index[1], n + ar % (npad - n)])
    src2d = src_full.reshape(nrows, _IB)
    dst2d = dst_full.reshape(nrows, _IB)

    w2 = jnp.concatenate([W_mu, W_ls], axis=1)
    b1r = b1.reshape(1, d_hid)
    b2r = jnp.concatenate([b_mu, b_ls]).reshape(1, 2 * d_out)

    degp = _make_deg(nrows, npad)(dst2d)
    degp3 = degp[:, :n].reshape(_NC, n, 1)

    r = 1000 if n % 1000 == 0 else 8 * (n // 8)
    grid = (n // r,)
    f32 = jnp.float32

    hs1, dis = pl.pallas_call(
        _b1_body,
        grid=grid,
        in_specs=[
            pl.BlockSpec((_NC, r, 1), lambda i: (0, i, 0)),
            pl.BlockSpec((r, d_in), lambda i: (i, 0)),
            pl.BlockSpec((d_in, d_hid), lambda i: (0, 0)),
        ],
        out_specs=[
            pl.BlockSpec((r, d_hid), lambda i: (i, 0)),
            pl.BlockSpec((r, 1), lambda i: (i, 0)),
        ],
        out_shape=[
            jax.ShapeDtypeStruct((n, d_hid), f32),
            jax.ShapeDtypeStruct((n, 1), f32),
        ],
    )(degp3, x, W1)

    pp = _make_agg(npad, d_hid, nrows)(hs1, src2d, dst2d)[:, :n]

    hs2 = pl.pallas_call(
        _b2_body,
        grid=grid,
        in_specs=[
            pl.BlockSpec((_NC, r, d_hid), lambda i: (0, i, 0)),
            pl.BlockSpec((r, 1), lambda i: (i, 0)),
            pl.BlockSpec((r, d_hid), lambda i: (i, 0)),
            pl.BlockSpec((d_hid, 2 * d_out), lambda i: (0, 0)),
            pl.BlockSpec((1, d_hid), lambda i: (0, 0)),
        ],
        out_specs=pl.BlockSpec((r, 2 * d_out), lambda i: (i, 0)),
        out_shape=jax.ShapeDtypeStruct((n, 2 * d_out), f32),
    )(pp, dis, hs1, w2, b1r)

    qq = _make_agg(npad, 2 * d_out, nrows)(hs2, src2d, dst2d)[:, :n]

    out = pl.pallas_call(
        _b3_body,
        grid=grid,
        in_specs=[
            pl.BlockSpec((_NC, r, 2 * d_out), lambda i: (0, i, 0)),
            pl.BlockSpec((r, 1), lambda i: (i, 0)),
            pl.BlockSpec((r, 2 * d_out), lambda i: (i, 0)),
            pl.BlockSpec((1, 2 * d_out), lambda i: (0, 0)),
        ],
        out_specs=pl.BlockSpec((r, 2 * d_out), lambda i: (i, 0)),
        out_shape=jax.ShapeDtypeStruct((n, 2 * d_out), f32),
    )(qq, dis, hs2, b2r)

    return (out[:, :d_out], out[:, d_out:])


# speedup vs baseline: 38.5443x; 1.0594x over previous
import functools

import jax
import jax.numpy as jnp
from jax import lax
from jax.experimental import pallas as pl
from jax.experimental.pallas import tpu as pltpu
from jax.experimental.pallas import tpu_sc as plsc

_NC = 2
_NS = 16
_NW = _NC * _NS
_IB = 128


_NBUF = 2


def _deg_body(dst2d, out, didx, ones_v, zbuf, accum):
    c = lax.axis_index("c")
    s = lax.axis_index("s")
    wid = c * _NS + s
    npad = accum.shape[0]
    rpt = npad // _NS
    rows = dst2d.shape[0] // _NW

    def fill_ones(i, _):
        ones_v[pl.ds(i * 16, 16)] = jnp.ones((16,), jnp.float32)
        return 0

    lax.fori_loop(0, _IB // 16, fill_ones, 0)

    def fill_zero(i, _):
        zbuf[pl.ds(i * 16, 16)] = jnp.zeros((16,), jnp.float32)
        return 0

    lax.fori_loop(0, rpt // 16, fill_zero, 0)

    pltpu.sync_copy(zbuf, accum.at[pl.ds(s * rpt, rpt)])
    pltpu.sync_copy(dst2d.at[pl.ds(wid * rows, rows)], didx)
    plsc.subcore_barrier()

    def body(j, _):
        pltpu.sync_copy(ones_v, accum.at[didx.at[j]], add=True)
        return 0

    lax.fori_loop(0, rows, body, 0)
    plsc.subcore_barrier()
    pltpu.sync_copy(accum.at[pl.ds(s * rpt, rpt)], out.at[c, pl.ds(s * rpt, rpt)])


def _make_deg(nrows, npad):
    mesh = plsc.VectorSubcoreMesh(core_axis_name="c", subcore_axis_name="s")
    return pl.kernel(
        _deg_body,
        out_type=jax.ShapeDtypeStruct((_NC, npad), jnp.float32),
        mesh=mesh,
        scratch_types=[
            pltpu.VMEM((nrows // _NW, _IB), jnp.int32),
            pltpu.VMEM((_IB,), jnp.float32),
            pltpu.VMEM((npad // _NS,), jnp.float32),
            pltpu.VMEM_SHARED((npad,), jnp.float32),
        ],
    )


_CH = 16


def _agg_body(hs, src2d, dst2d, out, sc0, sc1, dc0, dc1, b0, b1, accum, s0, s1):
    c = lax.axis_index("c")
    s = lax.axis_index("s")
    wid = c * _NS + s
    n = accum.shape[0]
    rpt = n // _NS
    rows = src2d.shape[0] // _NW
    nch = rows // _CH
    base = wid * rows
    scb = (sc0, sc1)
    dcb = (dc0, dc1)
    bufs = (b0, b1)
    sems = (s0, s1)

    def zfill(i, _):
        for cb in range(8):
            b0[i, pl.ds(cb * 16, 16)] = jnp.zeros((16,), jnp.float32)
        return 0

    lax.fori_loop(0, _IB, zfill, 0)
    for i in range(rpt // _IB):
        pltpu.sync_copy(b0, accum.at[pl.ds(s * rpt + i * _IB, _IB)])
    plsc.subcore_barrier()

    pltpu.sync_copy(src2d.at[pl.ds(base, _CH)], sc0)
    pltpu.sync_copy(dst2d.at[pl.ds(base, _CH)], dc0)
    pltpu.async_copy(hs.at[sc0.at[0]], b0, s0)
    pltpu.async_copy(hs.at[sc0.at[1]], b1, s1)

    for k in range(nch):
        cur_s, cur_d = scb[k % 2], dcb[k % 2]
        nxt_s, nxt_d = scb[(k + 1) % 2], dcb[(k + 1) % 2]
        if k + 1 < nch:
            pltpu.sync_copy(src2d.at[pl.ds(base + (k + 1) * _CH, _CH)], nxt_s)
            pltpu.sync_copy(dst2d.at[pl.ds(base + (k + 1) * _CH, _CH)], nxt_d)
        for jj in range(_CH):
            j = k * _CH + jj
            b = jj % 2
            pltpu.make_async_copy(hs.at[cur_s.at[jj]], bufs[b], sems[b]).wait()
            pltpu.sync_copy(bufs[b], accum.at[cur_d.at[jj]], add=True)
            nj = jj + 2
            if j + 2 < rows:
                if nj < _CH:
                    pltpu.async_copy(hs.at[cur_s.at[nj]], bufs[b], sems[b])
                else:
                    pltpu.async_copy(hs.at[nxt_s.at[nj - _CH]], bufs[b], sems[b])

    plsc.subcore_barrier()
    pltpu.sync_copy(
        accum.at[pl.ds(s * rpt, rpt)], out.at[c, pl.ds(s * rpt, rpt)]
    )


def _make_agg(npad, d, nrows):
    mesh = plsc.VectorSubcoreMesh(core_axis_name="c", subcore_axis_name="s")
    return pl.kernel(
        _agg_body,
        out_type=jax.ShapeDtypeStruct((_NC, npad, d), jnp.float32),
        mesh=mesh,
        scratch_types=[
            pltpu.VMEM((_CH, _IB), jnp.int32),
            pltpu.VMEM((_CH, _IB), jnp.int32),
            pltpu.VMEM((_CH, _IB), jnp.int32),
            pltpu.VMEM((_CH, _IB), jnp.int32),
            pltpu.VMEM((_IB, d), jnp.float32),
            pltpu.VMEM((_IB, d), jnp.float32),
            pltpu.VMEM_SHARED((npad, d), jnp.float32),
            pltpu.SemaphoreType.DMA,
            pltpu.SemaphoreType.DMA,
        ],
    )


def _b1_body(degp_ref, x_ref, w1_ref, hs_ref, dis_ref):
    deg = degp_ref[0] + degp_ref[1] + 1.0
    dis = lax.rsqrt(deg)
    h1 = jnp.dot(x_ref[...], w1_ref[...], preferred_element_type=jnp.float32)
    hs_ref[...] = h1 * dis
    dis_ref[...] = dis


def _b2_body(pp_ref, dis_ref, hs1_ref, w2_ref, b1_ref, hs2_ref):
    dis = dis_ref[...]
    agg = pp_ref[0] + pp_ref[1]
    h = jnp.maximum(dis * agg + dis * hs1_ref[...] + b1_ref[...], 0.0)
    h2 = jnp.dot(h, w2_ref[...], preferred_element_type=jnp.float32)
    hs2_ref[...] = h2 * dis


def _b3_body(qq_ref, dis_ref, hs2_ref, b2_ref, mu_ref, ls_ref):
    dis = dis_ref[...]
    out = dis * (qq_ref[0] + qq_ref[1]) + dis * hs2_ref[...] + b2_ref[...]
    d = mu_ref.shape[1]
    mu_ref[...] = out[:, :d]
    ls_ref[...] = out[:, d:]


def kernel(x, edge_index, W1, b1, W_mu, b_mu, W_ls, b_ls):
    n, d_in = x.shape
    e = edge_index.shape[1]
    d_hid = W1.shape[1]
    d_out = W_mu.shape[1]

    npad = ((n + 255) // 256) * 256

    rows_pt = 8 * (-(-e // (_IB * _NW * 8)))
    nrows = _NW * rows_pt
    pad = nrows * _IB - e
    ar = jnp.arange(pad, dtype=jnp.int32)
    src_full = jnp.concatenate([edge_index[0], ar % n])
    dst_full = jnp.concatenate([edge_index[1], n + ar % (npad - n)])
    src2d = src_full.reshape(nrows, _IB)
    dst2d = dst_full.reshape(nrows, _IB)

    w2 = jnp.concatenate([W_mu, W_ls], axis=1)
    b1r = b1.reshape(1, d_hid)
    b2r = jnp.concatenate([b_mu, b_ls]).reshape(1, 2 * d_out)

    degp = _make_deg(nrows, npad)(dst2d)
    degp3 = degp[:, :n].reshape(_NC, n, 1)

    r = 1000 if n % 1000 == 0 else 8 * (n // 8)
    grid = (n // r,)
    f32 = jnp.float32

    hs1, dis = pl.pallas_call(
        _b1_body,
        grid=grid,
        in_specs=[
            pl.BlockSpec((_NC, r, 1), lambda i: (0, i, 0)),
            pl.BlockSpec((r, d_in), lambda i: (i, 0)),
            pl.BlockSpec((d_in, d_hid), lambda i: (0, 0)),
        ],
        out_specs=[
            pl.BlockSpec((r, d_hid), lambda i: (i, 0)),
            pl.BlockSpec((r, 1), lambda i: (i, 0)),
        ],
        out_shape=[
            jax.ShapeDtypeStruct((n, d_hid), f32),
            jax.ShapeDtypeStruct((n, 1), f32),
        ],
    )(degp3, x, W1)

    pp = _make_agg(npad, d_hid, nrows)(hs1, src2d, dst2d)

    hs2 = pl.pallas_call(
        _b2_body,
        grid=grid,
        in_specs=[
            pl.BlockSpec((_NC, r, d_hid), lambda i: (0, i, 0)),
            pl.BlockSpec((r, 1), lambda i: (i, 0)),
            pl.BlockSpec((r, d_hid), lambda i: (i, 0)),
            pl.BlockSpec((d_hid, 2 * d_out), lambda i: (0, 0)),
            pl.BlockSpec((1, d_hid), lambda i: (0, 0)),
        ],
        out_specs=pl.BlockSpec((r, 2 * d_out), lambda i: (i, 0)),
        out_shape=jax.ShapeDtypeStruct((n, 2 * d_out), f32),
    )(pp, dis, hs1, w2, b1r)

    qq = _make_agg(npad, 2 * d_out, nrows)(hs2, src2d, dst2d)

    mu, ls = pl.pallas_call(
        _b3_body,
        grid=grid,
        in_specs=[
            pl.BlockSpec((_NC, r, 2 * d_out), lambda i: (0, i, 0)),
            pl.BlockSpec((r, 1), lambda i: (i, 0)),
            pl.BlockSpec((r, 2 * d_out), lambda i: (i, 0)),
            pl.BlockSpec((1, 2 * d_out), lambda i: (0, 0)),
        ],
        out_specs=[
            pl.BlockSpec((r, d_out), lambda i: (i, 0)),
            pl.BlockSpec((r, d_out), lambda i: (i, 0)),
        ],
        out_shape=[
            jax.ShapeDtypeStruct((n, d_out), f32),
            jax.ShapeDtypeStruct((n, d_out), f32),
        ],
    )(qq, dis, hs2, b2r)

    return (mu, ls)
